# baseline (speedup 1.0000x reference)
import functools

import jax
import jax.numpy as jnp
from jax import lax
from jax.experimental import pallas as pl
from jax.experimental.pallas import tpu as pltpu
from jax.experimental.pallas import tpu_sc as plsc

CHANS, SEQ, FEAT = 8, 4096, 1024
N_ROWS = SEQ // 4
ROWS = CHANS * SEQ
LANES = 16
NW = 32
SEG = SEQ // NW
CHUNK_R = 64
VECS = FEAT // LANES


def _sc_stream_body(x_hbm, idxs_hbm, warp_hbm, out_hbm, idx_v, warp_v, scale_v, buf, sem):
    wid = lax.axis_index("s") * 2 + lax.axis_index("c")
    seg_lo = wid * SEG

    pltpu.sync_copy(idxs_hbm, idx_v)
    pltpu.sync_copy(warp_hbm, warp_v)

    def init_body(i, carry):
        scale_v[pl.ds(i * LANES, LANES)] = jnp.full((LANES,), 1.0, jnp.float32)
        return carry

    lax.fori_loop(0, SEG // LANES, init_body, 0)

    def scat_body(i, carry):
        idx_chunk = idx_v[pl.ds(i * LANES, LANES)]
        w_chunk = warp_v[pl.ds(i * LANES, LANES)]
        local = idx_chunk - seg_lo
        mask = (local >= 0) & (local < SEG)
        plsc.store_scatter(scale_v, [local], w_chunk, mask=mask)
        return carry

    lax.fori_loop(0, N_ROWS // LANES, scat_body, 0)

    def chunk_body(k, carry):
        c = k // (SEG // CHUNK_R)
        o = k % (SEG // CHUNK_R)
        row_lo = c * SEQ + seg_lo + o * CHUNK_R
        pltpu.sync_copy(x_hbm.at[pl.ds(row_lo, CHUNK_R)], buf)

        def row_body(r, rcarry):
            splat = plsc.load_gather(
                scale_v, [jnp.full((LANES,), o * CHUNK_R, jnp.int32) + r]
            )

            def vec_body(j, vcarry):
                buf[r, pl.ds(j * LANES, LANES)] = (
                    buf[r, pl.ds(j * LANES, LANES)] * splat
                )
                return vcarry

            lax.fori_loop(0, VECS, vec_body, 0)
            return rcarry

        lax.fori_loop(0, CHUNK_R, row_body, 0)
        pltpu.sync_copy(buf, out_hbm.at[pl.ds(row_lo, CHUNK_R)])
        return carry

    lax.fori_loop(0, CHANS * (SEG // CHUNK_R), chunk_body, 0)


_sc_stream = functools.partial(
    pl.kernel,
    out_type=jax.ShapeDtypeStruct((ROWS, FEAT), jnp.float32),
    mesh=plsc.VectorSubcoreMesh(core_axis_name="c", subcore_axis_name="s"),
    scratch_types=[
        pltpu.VMEM((N_ROWS,), jnp.int32),
        pltpu.VMEM((N_ROWS,), jnp.float32),
        pltpu.VMEM((SEG,), jnp.float32),
        pltpu.VMEM((CHUNK_R, FEAT), jnp.float32),
        pltpu.SemaphoreType.DMA,
    ],
    compiler_params=pltpu.CompilerParams(needs_layout_passes=False),
)(_sc_stream_body)


def kernel(x, idxs, warp):
    out2d = _sc_stream(x.reshape(ROWS, FEAT), idxs, warp.reshape(N_ROWS))
    return out2d.reshape(CHANS, SEQ, FEAT)
